# baseline (speedup 1.0000x reference)
import functools

import jax
import jax.numpy as jnp
from jax import lax
from jax.experimental import pallas as pl
from jax.experimental.pallas import tpu as pltpu
from jax.experimental.pallas import tpu_sc as plsc

_BLK = 128


def _build(n, k):
    info = plsc.get_sparse_core_info()
    nc, ns, lanes = info.num_cores, info.num_subcores, info.num_lanes
    nw = nc * ns
    assert n % _BLK == 0
    vpb = _BLK // lanes
    blocks_total = n // _BLK
    base_nb = blocks_total // nw
    extra = blocks_total % nw
    pb = 16
    max_nb = base_nb + (1 if extra else 0)
    assert base_nb >= pb
    npieces = -(-max_nb // pb)
    if npieces % 2:
        npieces += 1
    kstride = 2 * k
    crep_r = 8
    cs8 = kstride * crep_r
    halves = []
    h = k // 2
    while h >= 1:
        halves.append(h)
        h //= 2

    mesh = plsc.VectorSubcoreMesh(core_axis_name="c", subcore_axis_name="s")

    @functools.partial(
        pl.kernel,
        mesh=mesh,
        compiler_params=pltpu.CompilerParams(needs_layout_passes=False),
        out_type=jax.ShapeDtypeStruct((n,), jnp.float32),
        scratch_types=[
            pltpu.VMEM((2 * k * lanes,), jnp.float32),
            pltpu.VMEM((4 * cs8,), jnp.float32),
            pltpu.VMEM((pb * 2 * _BLK,), jnp.float32),
            pltpu.VMEM((pb * 2 * _BLK,), jnp.float32),
            pltpu.VMEM((pb * _BLK,), jnp.float32),
            pltpu.VMEM((pb * _BLK,), jnp.float32),
            pltpu.SemaphoreType.DMA,
            pltpu.SemaphoreType.DMA,
            pltpu.SemaphoreType.DMA,
            pltpu.SemaphoreType.DMA,
        ],
    )
    def run(xv_hbm, krep, crep, out, krep_v, crep_v,
            xb0, xb1, ob0, ob1, si0, si1, so0, so1):
        xbufs, obufs = (xb0, xb1), (ob0, ob1)
        sins, souts = (si0, si1), (so0, so1)
        wid = lax.axis_index("s") * nc + lax.axis_index("c")
        pltpu.sync_copy(krep, krep_v)
        pltpu.sync_copy(crep, crep_v)
        nb = base_nb + jnp.where(wid < extra, 1, 0)
        sb = wid * base_nb + jnp.minimum(wid, extra)
        iota = lax.iota(jnp.int32, lanes)
        and7 = jnp.bitwise_and(iota, 7)
        coff = [and7, and7 + (k - 1) * crep_r]

        def search(xv, d):
            bl16 = iota + d * (k * lanes)
            for half in halves:
                km = plsc.load_gather(krep_v, [bl16 + (half - 1) * lanes])
                bl16 = bl16 + jnp.where(km < xv, half * lanes, 0)
            return bl16

        def poly(bl16, d, xv):
            q = jnp.bitwise_and(bl16, -lanes)
            ci8 = lax.shift_right_logical(
                jnp.maximum(q - (d * (k * lanes) + lanes), 0), 1
            )
            ci8 = ci8 + coff[d]
            av = plsc.load_gather(crep_v, [ci8])
            bv = plsc.load_gather(crep_v, [ci8 + cs8])
            cv = plsc.load_gather(crep_v, [ci8 + 2 * cs8])
            dv = plsc.load_gather(crep_v, [ci8 + 3 * cs8])
            return ((dv * xv + cv) * xv + bv) * xv + av

        def compute(xb, ob):
            def blk(j, _):
                xbase = j * (2 * _BLK)
                obase = j * _BLK
                for u in range(vpb):
                    x0 = xb[pl.ds(xbase + u * lanes, lanes)]
                    x1 = xb[pl.ds(xbase + _BLK + u * lanes, lanes)]
                    p0 = poly(search(x0, 0), 0, x0)
                    p1 = poly(search(x1, 1), 1, x1)
                    ob[pl.ds(obase + u * lanes, lanes)] = p0 * p1
                return 0

            lax.fori_loop(0, pb, blk, 0)

        def off_of(p):
            return sb + jnp.minimum(p * pb, nb - pb)

        def issue_in(p, b):
            pltpu.async_copy(
                xv_hbm.at[pl.ds(off_of(p) * 2 * _BLK, pb * 2 * _BLK)],
                xbufs[b], sins[b],
            )

        issue_in(0, 0)

        def outer(io, _):
            for b in range(2):
                p = io * 2 + b

                @pl.when(p + 1 < npieces)
                def _():
                    issue_in(p + 1, 1 - b)

                pltpu.make_async_copy(
                    xv_hbm.at[pl.ds(0, pb * 2 * _BLK)], xbufs[b], sins[b]
                ).wait()

                @pl.when(p >= 2)
                def _():
                    pltpu.make_async_copy(
                        obufs[b], out.at[pl.ds(0, pb * _BLK)], souts[b]
                    ).wait()

                compute(xbufs[b], obufs[b])
                pltpu.async_copy(
                    obufs[b], out.at[pl.ds(off_of(p) * _BLK, pb * _BLK)],
                    souts[b],
                )
            return 0

        lax.fori_loop(0, npieces // 2, outer, 0)
        for b in range(2):
            pltpu.make_async_copy(
                obufs[b], out.at[pl.ds(0, pb * _BLK)], souts[b]
            ).wait()

    return run


@functools.cache
def _get(n, k):
    return _build(n, k)


def kernel(x, knots, a, b, c, d):
    n = x.shape[0]
    k = knots.shape[0]
    lanes = 16
    xv = x.reshape(n // _BLK, _BLK, 2).transpose(0, 2, 1).reshape(-1)
    krep = jnp.broadcast_to(knots.T[:, :, None], (2, k, lanes)).reshape(-1)
    coefs = jnp.stack(
        [a.T.reshape(-1), b.T.reshape(-1), c.T.reshape(-1), d.T.reshape(-1)]
    )
    ct = jnp.zeros((4, 2 * k), jnp.float32).at[:, : 2 * (k - 1)].set(coefs)
    crep = jnp.broadcast_to(ct[:, :, None], (4, 2 * k, 8)).reshape(-1)
    return _get(n, k)(xv, krep, crep)


# speedup vs baseline: 1.2018x; 1.2018x over previous
import functools

import jax
import jax.numpy as jnp
from jax import lax
from jax.experimental import pallas as pl
from jax.experimental.pallas import tpu as pltpu
from jax.experimental.pallas import tpu_sc as plsc

_BLK = 128


def _build(n, k):
    info = plsc.get_sparse_core_info()
    nc, ns, lanes = info.num_cores, info.num_subcores, info.num_lanes
    nw = nc * ns
    assert n % _BLK == 0
    vpb = _BLK // lanes
    blocks_total = n // _BLK
    base_nb = blocks_total // nw
    extra = blocks_total % nw
    pb = 16
    max_nb = base_nb + (1 if extra else 0)
    assert base_nb >= pb
    npieces = -(-max_nb // pb)
    if npieces % 2:
        npieces += 1
    kstride = 2 * k
    crep_r = 8
    cs8 = kstride * crep_r
    nsel = 4
    halves = []
    h = k // 2
    while h >= 1:
        halves.append(h)
        h //= 2
    sel_halves, gat_halves = halves[:nsel], halves[nsel:]

    mesh = plsc.VectorSubcoreMesh(core_axis_name="c", subcore_axis_name="s")

    @functools.partial(
        pl.kernel,
        mesh=mesh,
        compiler_params=pltpu.CompilerParams(needs_layout_passes=False),
        out_type=jax.ShapeDtypeStruct((n,), jnp.float32),
        scratch_types=[
            pltpu.VMEM((2 * k * lanes,), jnp.float32),
            pltpu.VMEM((4 * cs8,), jnp.float32),
            pltpu.VMEM((pb * 2 * _BLK,), jnp.float32),
            pltpu.VMEM((pb * 2 * _BLK,), jnp.float32),
            pltpu.VMEM((pb * _BLK,), jnp.float32),
            pltpu.VMEM((pb * _BLK,), jnp.float32),
            pltpu.SemaphoreType.DMA,
            pltpu.SemaphoreType.DMA,
            pltpu.SemaphoreType.DMA,
            pltpu.SemaphoreType.DMA,
        ],
    )
    def run(xv_hbm, krep, crep, out, krep_v, crep_v,
            xb0, xb1, ob0, ob1, si0, si1, so0, so1):
        xbufs, obufs = (xb0, xb1), (ob0, ob1)
        sins, souts = (si0, si1), (so0, so1)
        wid = lax.axis_index("s") * nc + lax.axis_index("c")
        pltpu.sync_copy(krep, krep_v)
        pltpu.sync_copy(crep, crep_v)
        nb = base_nb + jnp.where(wid < extra, 1, 0)
        sb = wid * base_nb + jnp.minimum(wid, extra)
        iota = lax.iota(jnp.int32, lanes)
        and7 = jnp.bitwise_and(iota, 7)

        def splat(idx):
            return plsc.load_gather(
                krep_v, [jnp.full((lanes,), idx * lanes, jnp.int32)]
            )

        sel_vals = []
        for d in (0, 1):
            per_level = []
            bits = []
            for half in sel_halves:
                cands = []
                for combo in range(1 << len(bits)):
                    base = sum(b for i, b in enumerate(bits) if (combo >> i) & 1)
                    cands.append(splat(d * k + base + half - 1))
                per_level.append(cands)
                bits.append(half)
            sel_vals.append(per_level)

        gat_idx = []
        for d in (0, 1):
            gat_idx.append(
                [iota + ((half - 1) + d * k) * lanes for half in gat_halves]
            )
        coff = [and7, and7 + (k - 1) * crep_r]

        def search(xv, d):
            preds = []
            bl16 = jnp.zeros((lanes,), jnp.int32)
            for j, half in enumerate(sel_halves):
                cands = sel_vals[d][j]
                for p in reversed(preds):
                    cands = [
                        jnp.where(p, hi, lo)
                        for lo, hi in zip(cands[: len(cands) // 2],
                                          cands[len(cands) // 2:])
                    ]
                pr = cands[0] < xv
                preds.append(pr)
                bl16 = bl16 + jnp.where(pr, half * lanes, 0)
            for i, half in enumerate(gat_halves):
                km = plsc.load_gather(krep_v, [bl16 + gat_idx[d][i]])
                bl16 = bl16 + jnp.where(km < xv, half * lanes, 0)
            return bl16

        def poly(bl16, d, xv):
            ci8 = lax.shift_right_logical(jnp.maximum(bl16 - lanes, 0), 1)
            ci8 = ci8 + coff[d]
            av = plsc.load_gather(crep_v, [ci8])
            bv = plsc.load_gather(crep_v, [ci8 + cs8])
            cv = plsc.load_gather(crep_v, [ci8 + 2 * cs8])
            dv = plsc.load_gather(crep_v, [ci8 + 3 * cs8])
            return ((dv * xv + cv) * xv + bv) * xv + av

        def compute(xb, ob):
            def blk(j, _):
                xbase = j * (2 * _BLK)
                obase = j * _BLK
                for u in range(vpb):
                    x0 = xb[pl.ds(xbase + u * lanes, lanes)]
                    x1 = xb[pl.ds(xbase + _BLK + u * lanes, lanes)]
                    p0 = poly(search(x0, 0), 0, x0)
                    p1 = poly(search(x1, 1), 1, x1)
                    ob[pl.ds(obase + u * lanes, lanes)] = p0 * p1
                return 0

            lax.fori_loop(0, pb, blk, 0)

        def off_of(p):
            return sb + jnp.minimum(p * pb, nb - pb)

        def issue_in(p, b):
            pltpu.async_copy(
                xv_hbm.at[pl.ds(off_of(p) * 2 * _BLK, pb * 2 * _BLK)],
                xbufs[b], sins[b],
            )

        issue_in(0, 0)

        def outer(io, _):
            for b in range(2):
                p = io * 2 + b

                @pl.when(p + 1 < npieces)
                def _():
                    issue_in(p + 1, 1 - b)

                pltpu.make_async_copy(
                    xv_hbm.at[pl.ds(0, pb * 2 * _BLK)], xbufs[b], sins[b]
                ).wait()

                @pl.when(p >= 2)
                def _():
                    pltpu.make_async_copy(
                        obufs[b], out.at[pl.ds(0, pb * _BLK)], souts[b]
                    ).wait()

                compute(xbufs[b], obufs[b])
                pltpu.async_copy(
                    obufs[b], out.at[pl.ds(off_of(p) * _BLK, pb * _BLK)],
                    souts[b],
                )
            return 0

        lax.fori_loop(0, npieces // 2, outer, 0)
        for b in range(2):
            pltpu.make_async_copy(
                obufs[b], out.at[pl.ds(0, pb * _BLK)], souts[b]
            ).wait()

    return run


@functools.cache
def _get(n, k):
    return _build(n, k)


def kernel(x, knots, a, b, c, d):
    n = x.shape[0]
    k = knots.shape[0]
    lanes = 16
    xv = x.reshape(n // _BLK, _BLK, 2).transpose(0, 2, 1).reshape(-1)
    krep = jnp.broadcast_to(knots.T[:, :, None], (2, k, lanes)).reshape(-1)
    coefs = jnp.stack(
        [a.T.reshape(-1), b.T.reshape(-1), c.T.reshape(-1), d.T.reshape(-1)]
    )
    ct = jnp.zeros((4, 2 * k), jnp.float32).at[:, : 2 * (k - 1)].set(coefs)
    crep = jnp.broadcast_to(ct[:, :, None], (4, 2 * k, 8)).reshape(-1)
    return _get(n, k)(xv, krep, crep)


# speedup vs baseline: 1.4506x; 1.2071x over previous
import functools

import jax
import jax.numpy as jnp
from jax import lax
from jax.experimental import pallas as pl
from jax.experimental.pallas import tpu as pltpu
from jax.experimental.pallas import tpu_sc as plsc

_BLK = 128


def _build(n, k):
    info = plsc.get_sparse_core_info()
    nc, ns, lanes = info.num_cores, info.num_subcores, info.num_lanes
    nw = nc * ns
    assert n % _BLK == 0
    vpb = _BLK // lanes
    blocks_total = n // _BLK
    base_nb = blocks_total // nw
    extra = blocks_total % nw
    pb = 16
    max_nb = base_nb + (1 if extra else 0)
    assert base_nb >= pb
    npieces = -(-max_nb // pb)
    if npieces % 2:
        npieces += 1
    kstride = 2 * k
    crep_r = 8
    cs8 = kstride * crep_r
    nsel = 4
    halves = []
    h = k // 2
    while h >= 1:
        halves.append(h)
        h //= 2
    sel_halves, gat_halves = halves[:nsel], halves[nsel:]

    mesh = plsc.VectorSubcoreMesh(core_axis_name="c", subcore_axis_name="s")

    @functools.partial(
        pl.kernel,
        mesh=mesh,
        compiler_params=pltpu.CompilerParams(needs_layout_passes=False),
        out_type=jax.ShapeDtypeStruct((n,), jnp.float32),
        scratch_types=[
            pltpu.VMEM((2 * k * lanes,), jnp.float32),
            pltpu.VMEM((4 * cs8,), jnp.float32),
            pltpu.VMEM((2, pb * _BLK), jnp.float32),
            pltpu.VMEM((2, pb * _BLK), jnp.float32),
            pltpu.VMEM((pb * _BLK,), jnp.float32),
            pltpu.VMEM((pb * _BLK,), jnp.float32),
            pltpu.SemaphoreType.DMA,
            pltpu.SemaphoreType.DMA,
            pltpu.SemaphoreType.DMA,
            pltpu.SemaphoreType.DMA,
        ],
    )
    def run(xv_hbm, krep, crep, out, krep_v, crep_v,
            xb0, xb1, ob0, ob1, si0, si1, so0, so1):
        xbufs, obufs = (xb0, xb1), (ob0, ob1)
        sins, souts = (si0, si1), (so0, so1)
        wid = lax.axis_index("s") * nc + lax.axis_index("c")
        pltpu.sync_copy(krep, krep_v)
        pltpu.sync_copy(crep, crep_v)
        nb = base_nb + jnp.where(wid < extra, 1, 0)
        sb = wid * base_nb + jnp.minimum(wid, extra)
        iota = lax.iota(jnp.int32, lanes)
        and7 = jnp.bitwise_and(iota, 7)

        def splat(idx):
            return plsc.load_gather(
                krep_v, [jnp.full((lanes,), idx * lanes, jnp.int32)]
            )

        sel_vals = []
        for d in (0, 1):
            per_level = []
            bits = []
            for half in sel_halves:
                cands = []
                for combo in range(1 << len(bits)):
                    base = sum(b for i, b in enumerate(bits) if (combo >> i) & 1)
                    cands.append(splat(d * k + base + half - 1))
                per_level.append(cands)
                bits.append(half)
            sel_vals.append(per_level)

        gat_idx = []
        for d in (0, 1):
            gat_idx.append(
                [iota + ((half - 1) + d * k) * lanes for half in gat_halves]
            )
        coff = [and7, and7 + (k - 1) * crep_r]

        def search(xv, d):
            preds = []
            bl16 = jnp.zeros((lanes,), jnp.int32)
            for j, half in enumerate(sel_halves):
                cands = sel_vals[d][j]
                for p in reversed(preds):
                    cands = [
                        jnp.where(p, hi, lo)
                        for lo, hi in zip(cands[: len(cands) // 2],
                                          cands[len(cands) // 2:])
                    ]
                pr = cands[0] < xv
                preds.append(pr)
                bl16 = bl16 + jnp.where(pr, half * lanes, 0)
            for i, half in enumerate(gat_halves):
                km = plsc.load_gather(krep_v, [bl16 + gat_idx[d][i]])
                bl16 = bl16 + jnp.where(km < xv, half * lanes, 0)
            return bl16

        def poly(bl16, d, xv):
            ci8 = lax.shift_right_logical(jnp.maximum(bl16 - lanes, 0), 1)
            ci8 = ci8 + coff[d]
            av = plsc.load_gather(crep_v, [ci8])
            bv = plsc.load_gather(crep_v, [ci8 + cs8])
            cv = plsc.load_gather(crep_v, [ci8 + 2 * cs8])
            dv = plsc.load_gather(crep_v, [ci8 + 3 * cs8])
            return ((dv * xv + cv) * xv + bv) * xv + av

        def compute(xb, ob):
            def blk(j, _):
                xbase = j * _BLK
                obase = j * _BLK
                for u in range(vpb):
                    x0 = xb[0, pl.ds(xbase + u * lanes, lanes)]
                    x1 = xb[1, pl.ds(xbase + u * lanes, lanes)]
                    p0 = poly(search(x0, 0), 0, x0)
                    p1 = poly(search(x1, 1), 1, x1)
                    ob[pl.ds(obase + u * lanes, lanes)] = p0 * p1
                return 0

            lax.fori_loop(0, pb, blk, 0)

        def off_of(p):
            return sb + jnp.minimum(p * pb, nb - pb)

        def issue_in(p, b):
            pltpu.async_copy(
                xv_hbm.at[:, pl.ds(off_of(p) * _BLK, pb * _BLK)],
                xbufs[b], sins[b],
            )

        issue_in(0, 0)

        def outer(io, _):
            for b in range(2):
                p = io * 2 + b

                @pl.when(p + 1 < npieces)
                def _():
                    issue_in(p + 1, 1 - b)

                pltpu.make_async_copy(
                    xv_hbm.at[:, pl.ds(0, pb * _BLK)], xbufs[b], sins[b]
                ).wait()

                @pl.when(p >= 2)
                def _():
                    pltpu.make_async_copy(
                        obufs[b], out.at[pl.ds(0, pb * _BLK)], souts[b]
                    ).wait()

                compute(xbufs[b], obufs[b])
                pltpu.async_copy(
                    obufs[b], out.at[pl.ds(off_of(p) * _BLK, pb * _BLK)],
                    souts[b],
                )
            return 0

        lax.fori_loop(0, npieces // 2, outer, 0)
        for b in range(2):
            pltpu.make_async_copy(
                obufs[b], out.at[pl.ds(0, pb * _BLK)], souts[b]
            ).wait()

    return run


@functools.cache
def _get(n, k):
    return _build(n, k)


def kernel(x, knots, a, b, c, d):
    n = x.shape[0]
    k = knots.shape[0]
    lanes = 16
    xv = x.T
    krep = jnp.broadcast_to(knots.T[:, :, None], (2, k, lanes)).reshape(-1)
    coefs = jnp.stack(
        [a.T.reshape(-1), b.T.reshape(-1), c.T.reshape(-1), d.T.reshape(-1)]
    )
    ct = jnp.zeros((4, 2 * k), jnp.float32).at[:, : 2 * (k - 1)].set(coefs)
    crep = jnp.broadcast_to(ct[:, :, None], (4, 2 * k, 8)).reshape(-1)
    return _get(n, k)(xv, krep, crep)
